# baseline (speedup 1.0000x reference)
import functools

import jax
import jax.numpy as jnp
from jax import lax
from jax.experimental import pallas as pl
from jax.experimental.pallas import tpu as pltpu
from jax.experimental.pallas import tpu_sc as plsc

DH = 16
NW = 32
K = 128

_MESH = plsc.VectorSubcoreMesh(core_axis_name="c", subcore_axis_name="s",
                               num_cores=2, num_subcores=16)
_SC_PARAMS = pltpu.CompilerParams(use_tc_tiling_on_sc=False)


NB = 8


def _segsum_body(C, RPT, hs, srct, dstt, zrows, out, src_v, dst_v, rows,
                 acc, gsem0, gsem1, ssem0, ssem1):
    c = lax.axis_index("c")
    s = lax.axis_index("s")
    wid = c * 16 + s
    gsem = (gsem0, gsem1)
    ssem = (ssem0, ssem1)
    rounds = C // NB

    pltpu.sync_copy(zrows, acc.at[pl.ds(s * RPT, RPT)])
    pltpu.sync_copy(srct.at[wid], src_v)
    pltpu.sync_copy(dstt.at[wid], dst_v)
    plsc.subcore_barrier()

    def fire_gathers(seti, base):
        for b in range(NB):
            pltpu.async_copy(hs.at[src_v.at[base + b]], rows.at[seti, b],
                             gsem[seti])

    def drain_gathers(seti):
        for b in range(NB):
            pltpu.make_async_copy(hs.at[pl.ds(0, K)], rows.at[seti, b],
                                  gsem[seti]).wait()

    def fire_scatters(seti, base):
        for b in range(NB):
            pltpu.async_copy(rows.at[seti, b], acc.at[dst_v.at[base + b]],
                             ssem[seti], add=True)

    def drain_scatters(seti):
        for b in range(NB):
            pltpu.make_async_copy(hs.at[pl.ds(0, K)], rows.at[seti, b],
                                  ssem[seti]).wait()

    fire_gathers(0, 0)
    fire_gathers(1, NB)

    def body(g, carry):
        for i in range(2):
            base = (2 * g + i) * NB
            drain_gathers(i)
            fire_scatters(i, base)
            drain_scatters(i)
            fire_gathers(i, base + 2 * NB)
        return carry

    lax.fori_loop(0, rounds // 2 - 1, body, 0)
    for i in range(2):
        base = (rounds - 2 + i) * NB
        drain_gathers(i)
        fire_scatters(i, base)
        drain_scatters(i)

    plsc.subcore_barrier()
    pltpu.sync_copy(acc.at[pl.ds(s * RPT, RPT)],
                    out.at[c, pl.ds(s * RPT, RPT)])


def _deg_body(C, RPT, dstt, zrows, ones, out, dst_v, ones_v, acc, ssem):
    c = lax.axis_index("c")
    s = lax.axis_index("s")
    wid = c * 16 + s
    pltpu.sync_copy(zrows, acc.at[pl.ds(s * RPT, RPT)])
    pltpu.sync_copy(dstt.at[wid], dst_v)
    pltpu.sync_copy(ones, ones_v)
    plsc.subcore_barrier()

    def fire(base):
        for b in range(8):
            pltpu.async_copy(ones_v, acc.at[dst_v.at[base + b]], ssem,
                             add=True)

    def drain():
        for b in range(8):
            pltpu.make_async_copy(zrows.at[pl.ds(0, K)], ones_v,
                                  ssem).wait()

    fire(0)

    def body(g, carry):
        fire(8 * (g + 1))
        drain()
        return carry

    lax.fori_loop(0, C // 8 - 1, body, 0)
    drain()

    plsc.subcore_barrier()
    pltpu.sync_copy(acc.at[pl.ds(s * RPT, RPT)],
                    out.at[c, pl.ds(s * RPT, RPT)])


def _sc_segsum(hs, srct, dstt, zrows, C, NP, RPT):
    return pl.kernel(
        functools.partial(_segsum_body, C, RPT),
        out_type=jax.ShapeDtypeStruct((2, NP, DH), jnp.float32),
        mesh=_MESH,
        scratch_types=[
            pltpu.VMEM((C, K), jnp.int32),
            pltpu.VMEM((C, K), jnp.int32),
            pltpu.VMEM((2, NB, K, DH), jnp.float32),
            pltpu.VMEM_SHARED((NP, DH), jnp.float32),
            pltpu.SemaphoreType.DMA,
            pltpu.SemaphoreType.DMA,
            pltpu.SemaphoreType.DMA,
            pltpu.SemaphoreType.DMA,
        ],
        compiler_params=_SC_PARAMS,
    )(hs, srct, dstt, zrows)


def _sc_deg(dstt, zrows, ones, C, NP, RPT):
    return pl.kernel(
        functools.partial(_deg_body, C, RPT),
        out_type=jax.ShapeDtypeStruct((2, NP, DH), jnp.float32),
        mesh=_MESH,
        scratch_types=[
            pltpu.VMEM((C, K), jnp.int32),
            pltpu.VMEM((K, DH), jnp.float32),
            pltpu.VMEM_SHARED((NP, DH), jnp.float32),
            pltpu.SemaphoreType.DMA,
        ],
        compiler_params=_SC_PARAMS,
    )(dstt, zrows, ones)


def _edge_body(e_total, n, eb, row, out):
    i = pl.program_id(0)
    g = i * eb + lax.broadcasted_iota(jnp.int32, (eb,), 0)
    valid = g < e_total
    pad = n + jnp.remainder(g - e_total, K)
    out[:] = jnp.where(valid, row[0, 0, :], pad)


def _dinv(deg):
    cnt = deg[0, :, 0:1] + deg[1, :, 0:1]
    return lax.rsqrt(cnt + 1.0)


def _tc1_body(deg, x, w1, hs_out):
    h = jnp.dot(x[:], w1[:], preferred_element_type=jnp.float32)
    hs_out[:] = h * _dinv(deg)


def _tc2_body(deg, agg, hs1, b1, w2, out):
    dv = _dinv(deg)
    t = dv * (agg[0] + agg[1] + hs1[:]) + b1[0:1, :]
    h = jnp.maximum(t, 0.0)
    out[:] = jnp.dot(h, w2[:], preferred_element_type=jnp.float32) * dv


def _tc3_body(deg, agg, hs2, b2, out):
    dv = _dinv(deg)
    t = dv * (agg[0] + agg[1] + hs2[:]) + b2[0:1, :]
    m = jnp.max(t, axis=1, keepdims=True)
    lse = jnp.log(jnp.sum(jnp.exp(t - m), axis=1, keepdims=True))
    out[:] = t - m - lse


def kernel(x, edge_index, W1, b1, W2, b2):
    n, di = x.shape
    e = edge_index.shape[1]
    c_chunks = 8 * (-(-e // (NW * K * 8)))
    epad = NW * c_chunks * K
    rpt = 8 * (-(-(n + K) // (16 * 8)))
    np_rows = 16 * rpt

    eb = epad // 8
    eidx3 = edge_index.reshape(2, 1, e)

    def _edge_build(r):
        return pl.pallas_call(
            functools.partial(_edge_body, e, n, eb),
            grid=(8,),
            in_specs=[pl.BlockSpec((1, 1, eb), lambda i, r=r: (r, 0, i))],
            out_specs=pl.BlockSpec((eb,), lambda i: (i,)),
            out_shape=jax.ShapeDtypeStruct((epad,), jnp.int32),
        )(eidx3)

    dstt = _edge_build(1).reshape(NW, c_chunks, K)
    srct = _edge_build(0).reshape(NW, c_chunks, K)
    zrows = jnp.zeros((rpt, DH), jnp.float32)
    ones = jnp.ones((K, DH), jnp.float32)

    deg = _sc_deg(dstt, zrows, ones, c_chunks, np_rows, rpt)

    blk = 2000
    grid = (n // blk,)
    row16 = lambda i: (i, 0)
    deg_spec = pl.BlockSpec((2, blk, DH), lambda i: (0, i, 0))
    agg_spec = deg_spec
    hs1 = pl.pallas_call(
        _tc1_body,
        grid=grid,
        in_specs=[
            deg_spec,
            pl.BlockSpec((blk, di), row16),
            pl.BlockSpec((di, DH), lambda i: (0, 0)),
        ],
        out_specs=pl.BlockSpec((blk, DH), row16),
        out_shape=jax.ShapeDtypeStruct((np_rows, DH), jnp.float32),
    )(deg, x, W1)

    agg1 = _sc_segsum(hs1, srct, dstt, zrows, c_chunks, np_rows, rpt)

    b1b = jnp.broadcast_to(b1.reshape(1, DH), (8, DH))
    b2b = jnp.broadcast_to(b2.reshape(1, DH), (8, DH))
    hs2 = pl.pallas_call(
        _tc2_body,
        grid=grid,
        in_specs=[
            deg_spec,
            agg_spec,
            pl.BlockSpec((blk, DH), row16),
            pl.BlockSpec((8, DH), lambda i: (0, 0)),
            pl.BlockSpec((DH, DH), lambda i: (0, 0)),
        ],
        out_specs=pl.BlockSpec((blk, DH), row16),
        out_shape=jax.ShapeDtypeStruct((np_rows, DH), jnp.float32),
    )(deg, agg1, hs1, b1b, W2)

    agg2 = _sc_segsum(hs2, srct, dstt, zrows, c_chunks, np_rows, rpt)

    out = pl.pallas_call(
        _tc3_body,
        grid=grid,
        in_specs=[
            deg_spec,
            agg_spec,
            pl.BlockSpec((blk, DH), row16),
            pl.BlockSpec((8, DH), lambda i: (0, 0)),
        ],
        out_specs=pl.BlockSpec((blk, DH), row16),
        out_shape=jax.ShapeDtypeStruct((n, DH), jnp.float32),
    )(deg, agg2, hs2, b2b)

    return out


# speedup vs baseline: 1.2561x; 1.2561x over previous
import functools

import jax
import jax.numpy as jnp
from jax import lax
from jax.experimental import pallas as pl
from jax.experimental.pallas import tpu as pltpu
from jax.experimental.pallas import tpu_sc as plsc

DH = 16
NW = 32
K = 128

_MESH = plsc.VectorSubcoreMesh(core_axis_name="c", subcore_axis_name="s",
                               num_cores=2, num_subcores=16)
_SC_PARAMS = pltpu.CompilerParams(use_tc_tiling_on_sc=False)


NB = 8


def _segsum_body(C, RPT, hs, srct, dstt, zrows, out, src_v, dst_v, rows,
                 acc, gsem0, gsem1, ssem0, ssem1):
    c = lax.axis_index("c")
    s = lax.axis_index("s")
    wid = c * 16 + s
    gsem = (gsem0, gsem1)
    ssem = (ssem0, ssem1)
    rounds = C // NB

    pltpu.sync_copy(zrows, acc.at[pl.ds(s * RPT, RPT)])
    pltpu.sync_copy(srct.at[wid], src_v)
    pltpu.sync_copy(dstt.at[wid], dst_v)
    plsc.subcore_barrier()

    def fire_gathers(seti, base):
        for b in range(NB):
            pltpu.async_copy(hs.at[src_v.at[base + b]], rows.at[seti, b],
                             gsem[seti])

    def drain_gathers(seti):
        for b in range(NB):
            pltpu.make_async_copy(hs.at[pl.ds(0, K)], rows.at[seti, b],
                                  gsem[seti]).wait()

    def fire_scatters(seti, base):
        for b in range(NB):
            pltpu.async_copy(rows.at[seti, b], acc.at[dst_v.at[base + b]],
                             ssem[seti], add=True)

    def drain_scatters(seti):
        for b in range(NB):
            pltpu.make_async_copy(hs.at[pl.ds(0, K)], rows.at[seti, b],
                                  ssem[seti]).wait()

    fire_gathers(0, 0)
    fire_gathers(1, NB)

    def body(g, carry):
        for i in range(2):
            base = (2 * g + i) * NB
            drain_gathers(i)
            fire_scatters(i, base)
            drain_scatters(i)
            fire_gathers(i, base + 2 * NB)
        return carry

    lax.fori_loop(0, rounds // 2 - 1, body, 0)
    for i in range(2):
        base = (rounds - 2 + i) * NB
        drain_gathers(i)
        fire_scatters(i, base)
        drain_scatters(i)

    plsc.subcore_barrier()
    pltpu.sync_copy(acc.at[pl.ds(s * RPT, RPT)],
                    out.at[c, pl.ds(s * RPT, RPT)])


def _deg_body(C, RPT, dstt, zrows, ones, out, dst_v, ones_v, acc, ssem):
    c = lax.axis_index("c")
    s = lax.axis_index("s")
    wid = c * 16 + s
    pltpu.sync_copy(zrows, acc.at[pl.ds(s * RPT, RPT)])
    pltpu.sync_copy(dstt.at[wid], dst_v)
    pltpu.sync_copy(ones, ones_v)
    plsc.subcore_barrier()

    def fire(base):
        for b in range(8):
            pltpu.async_copy(ones_v, acc.at[dst_v.at[base + b]], ssem,
                             add=True)

    def drain():
        for b in range(8):
            pltpu.make_async_copy(zrows.at[pl.ds(0, K)], ones_v,
                                  ssem).wait()

    fire(0)

    def body(g, carry):
        fire(8 * (g + 1))
        drain()
        return carry

    lax.fori_loop(0, C // 8 - 1, body, 0)
    drain()

    plsc.subcore_barrier()
    pltpu.sync_copy(acc.at[pl.ds(s * RPT, RPT)],
                    out.at[c, pl.ds(s * RPT, RPT)])


def _sc_segsum(hs, srct, dstt, zrows, C, NP, RPT):
    return pl.kernel(
        functools.partial(_segsum_body, C, RPT),
        out_type=jax.ShapeDtypeStruct((2, NP, DH), jnp.float32),
        mesh=_MESH,
        scratch_types=[
            pltpu.VMEM((C, K), jnp.int32),
            pltpu.VMEM((C, K), jnp.int32),
            pltpu.VMEM((2, NB, K, DH), jnp.float32),
            pltpu.VMEM_SHARED((NP, DH), jnp.float32),
            pltpu.SemaphoreType.DMA,
            pltpu.SemaphoreType.DMA,
            pltpu.SemaphoreType.DMA,
            pltpu.SemaphoreType.DMA,
        ],
        compiler_params=_SC_PARAMS,
    )(hs, srct, dstt, zrows)


def _sc_deg(dstt, zrows, ones, C, NP, RPT):
    return pl.kernel(
        functools.partial(_deg_body, C, RPT),
        out_type=jax.ShapeDtypeStruct((2, NP, DH), jnp.float32),
        mesh=_MESH,
        scratch_types=[
            pltpu.VMEM((C, K), jnp.int32),
            pltpu.VMEM((K, DH), jnp.float32),
            pltpu.VMEM_SHARED((NP, DH), jnp.float32),
            pltpu.SemaphoreType.DMA,
        ],
        compiler_params=_SC_PARAMS,
    )(dstt, zrows, ones)


def _edge_body(e_total, n, eb, eidx, s_out, d_out):
    i = pl.program_id(0)
    g = i * eb + lax.broadcasted_iota(jnp.int32, (eb,), 0)
    valid = g < e_total
    pad = n + jnp.remainder(g - e_total, K)
    s_out[:] = jnp.where(valid, eidx[0, :], pad)
    d_out[:] = jnp.where(valid, eidx[1, :], pad)


def _dinv(deg):
    cnt = deg[0, :, 0:1] + deg[1, :, 0:1]
    return lax.rsqrt(cnt + 1.0)


def _tc1_body(deg, x, w1, hs_out):
    h = jnp.dot(x[:], w1[:], preferred_element_type=jnp.float32)
    hs_out[:] = h * _dinv(deg)


def _tc2_body(deg, agg, hs1, b1, w2bd, out):
    cnt = deg[0] + deg[1]
    dv = lax.rsqrt(cnt + 1.0)
    t = dv * (agg[0] + agg[1] + hs1[:]) + b1[0:1, :]
    h = jnp.maximum(t, 0.0)
    out[:] = jnp.dot(h, w2bd[:], preferred_element_type=jnp.float32) * dv


def _tc3_body(deg, agg, hs2, b2, out):
    cnt = deg[0] + deg[1]
    dv = lax.rsqrt(cnt + 1.0)
    t = dv * (agg[0] + agg[1] + hs2[:]) + b2[0:1, :]
    ii = lax.broadcasted_iota(jnp.int32, (128, 128), 0)
    jj = lax.broadcasted_iota(jnp.int32, (128, 128), 1)
    sel = ((jj // DH) * DH == ii).astype(jnp.float32)
    gsum = (jj // DH == ii // DH).astype(jnp.float32)
    m = t
    for k in (1, 2, 4, 8):
        m = jnp.maximum(m, pltpu.roll(m, 128 - k, 1))
    mg = jnp.dot(m, sel, preferred_element_type=jnp.float32)
    sg = jnp.dot(jnp.exp(t - mg), gsum, preferred_element_type=jnp.float32)
    out[:] = t - mg - jnp.log(sg)


def kernel(x, edge_index, W1, b1, W2, b2):
    n, di = x.shape
    e = edge_index.shape[1]
    c_chunks = 8 * (-(-e // (NW * K * 8)))
    epad = NW * c_chunks * K
    rpt = 8 * (-(-(n + K) // (16 * 8)))
    np_rows = 16 * rpt

    eb = epad // 8
    sflat, dflat = pl.pallas_call(
        functools.partial(_edge_body, e, n, eb),
        grid=(8,),
        in_specs=[pl.BlockSpec((2, eb), lambda i: (0, i))],
        out_specs=[pl.BlockSpec((eb,), lambda i: (i,)),
                   pl.BlockSpec((eb,), lambda i: (i,))],
        out_shape=[jax.ShapeDtypeStruct((epad,), jnp.int32),
                   jax.ShapeDtypeStruct((epad,), jnp.int32)],
    )(edge_index)
    srct = sflat.reshape(NW, c_chunks, K)
    dstt = dflat.reshape(NW, c_chunks, K)
    zrows = jnp.zeros((rpt, DH), jnp.float32)
    ones = jnp.ones((K, DH), jnp.float32)

    deg = _sc_deg(dstt, zrows, ones, c_chunks, np_rows, rpt)

    blk = 2000
    grid = (n // blk,)
    row16 = lambda i: (i, 0)
    deg_spec = pl.BlockSpec((2, blk, DH), lambda i: (0, i, 0))
    agg_spec = deg_spec
    hs1 = pl.pallas_call(
        _tc1_body,
        grid=grid,
        in_specs=[
            deg_spec,
            pl.BlockSpec((blk, di), row16),
            pl.BlockSpec((di, DH), lambda i: (0, 0)),
        ],
        out_specs=pl.BlockSpec((blk, DH), row16),
        out_shape=jax.ShapeDtypeStruct((np_rows, DH), jnp.float32),
    )(deg, x, W1)

    agg1 = _sc_segsum(hs1, srct, dstt, zrows, c_chunks, np_rows, rpt)

    npk = np_rows // 8
    blkp = npk // grid[0]
    degp = deg.reshape(2, npk, 128)
    hs1p = hs1.reshape(npk, 128)
    agg1p = agg1.reshape(2, npk, 128)
    pk_spec = pl.BlockSpec((blkp, 128), row16)
    pk2_spec = pl.BlockSpec((2, blkp, 128), lambda i: (0, i, 0))
    b1p = jnp.broadcast_to(jnp.tile(b1, 8).reshape(1, 128), (8, 128))
    b2p = jnp.broadcast_to(jnp.tile(b2, 8).reshape(1, 128), (8, 128))
    w2bd = jnp.kron(jnp.eye(8, dtype=jnp.float32), W2)

    hs2p = pl.pallas_call(
        _tc2_body,
        grid=grid,
        in_specs=[
            pk2_spec,
            pk2_spec,
            pk_spec,
            pl.BlockSpec((8, 128), lambda i: (0, 0)),
            pl.BlockSpec((128, 128), lambda i: (0, 0)),
        ],
        out_specs=pk_spec,
        out_shape=jax.ShapeDtypeStruct((npk, 128), jnp.float32),
    )(degp, agg1p, hs1p, b1p, w2bd)

    agg2 = _sc_segsum(hs2p.reshape(np_rows, DH), srct, dstt, zrows,
                      c_chunks, np_rows, rpt)

    outp = pl.pallas_call(
        _tc3_body,
        grid=grid,
        in_specs=[
            pk2_spec,
            pk2_spec,
            pk_spec,
            pl.BlockSpec((8, 128), lambda i: (0, 0)),
        ],
        out_specs=pk_spec,
        out_shape=jax.ShapeDtypeStruct((npk, 128), jnp.float32),
    )(degp, agg2.reshape(2, npk, 128), hs2p, b2p)

    return outp.reshape(np_rows, DH)[:n]


# speedup vs baseline: 1.3684x; 1.0895x over previous
import functools

import jax
import jax.numpy as jnp
from jax import lax
from jax.experimental import pallas as pl
from jax.experimental.pallas import tpu as pltpu
from jax.experimental.pallas import tpu_sc as plsc

DH = 16
NW = 32
K = 128

_MESH = plsc.VectorSubcoreMesh(core_axis_name="c", subcore_axis_name="s",
                               num_cores=2, num_subcores=16)
_SC_PARAMS = pltpu.CompilerParams(use_tc_tiling_on_sc=False)


NB = 8


def _segsum_body(C, RPT, hs, srct, dstt, zrows, out, src_v, dst_v, rows,
                 acc, gsem0, gsem1, ssem0, ssem1):
    c = lax.axis_index("c")
    s = lax.axis_index("s")
    wid = c * 16 + s
    gsem = (gsem0, gsem1)
    ssem = (ssem0, ssem1)
    rounds = C // NB

    pltpu.sync_copy(zrows, acc.at[pl.ds(s * RPT, RPT)])
    pltpu.sync_copy(srct.at[wid], src_v)
    pltpu.sync_copy(dstt.at[wid], dst_v)
    plsc.subcore_barrier()

    def fire_gathers(seti, base):
        for b in range(NB):
            pltpu.async_copy(hs.at[src_v.at[base + b]], rows.at[seti, b],
                             gsem[seti])

    def drain_gathers(seti):
        for b in range(NB):
            pltpu.make_async_copy(hs.at[pl.ds(0, K)], rows.at[seti, b],
                                  gsem[seti]).wait()

    def fire_scatters(seti, base):
        for b in range(NB):
            pltpu.async_copy(rows.at[seti, b], acc.at[dst_v.at[base + b]],
                             ssem[seti], add=True)

    def drain_scatters(seti):
        for b in range(NB):
            pltpu.make_async_copy(hs.at[pl.ds(0, K)], rows.at[seti, b],
                                  ssem[seti]).wait()

    fire_gathers(0, 0)
    fire_gathers(1, NB)

    def body(g, carry):
        for i in range(2):
            base = (2 * g + i) * NB
            drain_gathers(i)
            fire_scatters(i, base)
            drain_scatters(i)
            fire_gathers(i, base + 2 * NB)
        return carry

    lax.fori_loop(0, rounds // 2 - 1, body, 0)
    for i in range(2):
        base = (rounds - 2 + i) * NB
        drain_gathers(i)
        fire_scatters(i, base)
        drain_scatters(i)

    plsc.subcore_barrier()
    pltpu.sync_copy(acc.at[pl.ds(s * RPT, RPT)],
                    out.at[c, pl.ds(s * RPT, RPT)])


def _deg_body(C, RPT, dstt, zrows, ones, out, dst_v, ones_v, acc, ssem):
    c = lax.axis_index("c")
    s = lax.axis_index("s")
    wid = c * 16 + s
    pltpu.sync_copy(zrows, acc.at[pl.ds(s * RPT, RPT)])
    pltpu.sync_copy(dstt.at[wid], dst_v)
    pltpu.sync_copy(ones, ones_v)
    plsc.subcore_barrier()

    def fire(base):
        for b in range(8):
            pltpu.async_copy(ones_v, acc.at[dst_v.at[base + b]], ssem,
                             add=True)

    def drain():
        for b in range(8):
            pltpu.make_async_copy(zrows.at[pl.ds(0, K)], ones_v,
                                  ssem).wait()

    fire(0)

    def body(g, carry):
        fire(8 * (g + 1))
        drain()
        return carry

    lax.fori_loop(0, C // 8 - 1, body, 0)
    drain()

    plsc.subcore_barrier()
    pltpu.sync_copy(acc.at[pl.ds(s * RPT, RPT)],
                    out.at[c, pl.ds(s * RPT, RPT)])


def _sc_segsum(hs, srct, dstt, zrows, C, NP, RPT):
    return pl.kernel(
        functools.partial(_segsum_body, C, RPT),
        out_type=jax.ShapeDtypeStruct((2, NP, DH), jnp.float32),
        mesh=_MESH,
        scratch_types=[
            pltpu.VMEM((C, K), jnp.int32),
            pltpu.VMEM((C, K), jnp.int32),
            pltpu.VMEM((2, NB, K, DH), jnp.float32),
            pltpu.VMEM_SHARED((NP, DH), jnp.float32),
            pltpu.SemaphoreType.DMA,
            pltpu.SemaphoreType.DMA,
            pltpu.SemaphoreType.DMA,
            pltpu.SemaphoreType.DMA,
        ],
        compiler_params=_SC_PARAMS,
    )(hs, srct, dstt, zrows)


def _sc_deg(dstt, zrows, ones, C, NP, RPT):
    return pl.kernel(
        functools.partial(_deg_body, C, RPT),
        out_type=jax.ShapeDtypeStruct((2, NP, DH), jnp.float32),
        mesh=_MESH,
        scratch_types=[
            pltpu.VMEM((C, K), jnp.int32),
            pltpu.VMEM((K, DH), jnp.float32),
            pltpu.VMEM_SHARED((NP, DH), jnp.float32),
            pltpu.SemaphoreType.DMA,
        ],
        compiler_params=_SC_PARAMS,
    )(dstt, zrows, ones)


def _edge_body(e_total, n, rb, w, eidx, s_out, d_out):
    i = pl.program_id(0)
    r = lax.broadcasted_iota(jnp.int32, (rb, w), 0)
    c = lax.broadcasted_iota(jnp.int32, (rb, w), 1)
    g = (i * rb + r) * w + c
    valid = g < e_total
    pad = n + jnp.remainder(g - e_total, K)
    s_out[:] = jnp.where(valid, eidx[0], pad)
    d_out[:] = jnp.where(valid, eidx[1], pad)


def _dinv(deg):
    cnt = deg[0, :, 0:1] + deg[1, :, 0:1]
    return lax.rsqrt(cnt + 1.0)


def _tc1_body(x, w1, h_out):
    h_out[:] = jnp.dot(x[:], w1[:], preferred_element_type=jnp.float32)


def _scale_body(deg, h1, out):
    cnt = deg[0] + deg[1]
    out[:] = h1[:] * lax.rsqrt(cnt + 1.0)


def _tc2_body(deg, agg, hs1, b1, w2bd, out):
    cnt = deg[0] + deg[1]
    dv = lax.rsqrt(cnt + 1.0)
    t = dv * (agg[0] + agg[1] + hs1[:]) + b1[0:1, :]
    h = jnp.maximum(t, 0.0)
    out[:] = jnp.dot(h, w2bd[:], preferred_element_type=jnp.float32) * dv


def _tc3_body(deg, agg, hs2, b2, out):
    cnt = deg[0] + deg[1]
    dv = lax.rsqrt(cnt + 1.0)
    t = dv * (agg[0] + agg[1] + hs2[:]) + b2[0:1, :]
    ii = lax.broadcasted_iota(jnp.int32, (128, 128), 0)
    jj = lax.broadcasted_iota(jnp.int32, (128, 128), 1)
    sel = ((jj // DH) * DH == ii).astype(jnp.float32)
    gsum = (jj // DH == ii // DH).astype(jnp.float32)
    m = t
    for k in (1, 2, 4, 8):
        m = jnp.maximum(m, pltpu.roll(m, 128 - k, 1))
    mg = jnp.dot(m, sel, preferred_element_type=jnp.float32)
    sg = jnp.dot(jnp.exp(t - mg), gsum, preferred_element_type=jnp.float32)
    out[:] = t - mg - jnp.log(sg)


def kernel(x, edge_index, W1, b1, W2, b2):
    n, di = x.shape
    e = edge_index.shape[1]
    c_chunks = 8 * (-(-e // (NW * K * 8)))
    epad = NW * c_chunks * K
    rpt = 8 * (-(-(n + K) // (16 * 8)))
    np_rows = 16 * rpt

    w = 512
    erows = -(-e // w)
    rb = epad // w // 10
    sflat, dflat = pl.pallas_call(
        functools.partial(_edge_body, e, n, rb, w),
        grid=(10,),
        in_specs=[pl.BlockSpec((2, rb, w), lambda i: (0, i, 0))],
        out_specs=[pl.BlockSpec((rb, w), lambda i: (i, 0)),
                   pl.BlockSpec((rb, w), lambda i: (i, 0))],
        out_shape=[jax.ShapeDtypeStruct((epad // w, w), jnp.int32),
                   jax.ShapeDtypeStruct((epad // w, w), jnp.int32)],
    )(edge_index.reshape(2, erows, w))
    srct = sflat.reshape(NW, c_chunks, K)
    dstt = dflat.reshape(NW, c_chunks, K)
    zrows = jnp.zeros((rpt, DH), jnp.float32)
    ones = jnp.ones((K, DH), jnp.float32)

    deg = _sc_deg(dstt, zrows, ones, c_chunks, np_rows, rpt)

    blk = 2000
    grid = (n // blk,)
    row16 = lambda i: (i, 0)
    h1 = pl.pallas_call(
        _tc1_body,
        grid=grid,
        in_specs=[
            pl.BlockSpec((blk, di), row16),
            pl.BlockSpec((di, DH), lambda i: (0, 0)),
        ],
        out_specs=pl.BlockSpec((blk, DH), row16),
        out_shape=jax.ShapeDtypeStruct((np_rows, DH), jnp.float32),
    )(x, W1)

    npk = np_rows // 8
    blkp = npk // grid[0]
    degp = deg.reshape(2, npk, 128)
    h1p = h1.reshape(npk, 128)
    pk_spec = pl.BlockSpec((blkp, 128), row16)
    pk2_spec = pl.BlockSpec((2, blkp, 128), lambda i: (0, i, 0))

    hs1p = pl.pallas_call(
        _scale_body,
        grid=grid,
        in_specs=[pk2_spec, pk_spec],
        out_specs=pk_spec,
        out_shape=jax.ShapeDtypeStruct((npk, 128), jnp.float32),
    )(degp, h1p)

    agg1 = _sc_segsum(hs1p.reshape(np_rows, DH), srct, dstt, zrows,
                      c_chunks, np_rows, rpt)
    agg1p = agg1.reshape(2, npk, 128)
    b1p = jnp.broadcast_to(jnp.tile(b1, 8).reshape(1, 128), (8, 128))
    b2p = jnp.broadcast_to(jnp.tile(b2, 8).reshape(1, 128), (8, 128))
    w2bd = jnp.kron(jnp.eye(8, dtype=jnp.float32), W2)

    hs2p = pl.pallas_call(
        _tc2_body,
        grid=grid,
        in_specs=[
            pk2_spec,
            pk2_spec,
            pk_spec,
            pl.BlockSpec((8, 128), lambda i: (0, 0)),
            pl.BlockSpec((128, 128), lambda i: (0, 0)),
        ],
        out_specs=pk_spec,
        out_shape=jax.ShapeDtypeStruct((npk, 128), jnp.float32),
    )(degp, agg1p, hs1p, b1p, w2bd)

    agg2 = _sc_segsum(hs2p.reshape(np_rows, DH), srct, dstt, zrows,
                      c_chunks, np_rows, rpt)

    outp = pl.pallas_call(
        _tc3_body,
        grid=grid,
        in_specs=[
            pk2_spec,
            pk2_spec,
            pk_spec,
            pl.BlockSpec((8, 128), lambda i: (0, 0)),
        ],
        out_specs=pk_spec,
        out_shape=jax.ShapeDtypeStruct((npk, 128), jnp.float32),
    )(degp, agg2.reshape(2, npk, 128), hs2p, b2p)

    return outp.reshape(np_rows, DH)[:n]


# speedup vs baseline: 1.4030x; 1.0253x over previous
import functools

import jax
import jax.numpy as jnp
from jax import lax
from jax.experimental import pallas as pl
from jax.experimental.pallas import tpu as pltpu
from jax.experimental.pallas import tpu_sc as plsc

DH = 16
NW = 32
K = 128

_MESH = plsc.VectorSubcoreMesh(core_axis_name="c", subcore_axis_name="s",
                               num_cores=2, num_subcores=16)
_SC_PARAMS = pltpu.CompilerParams(use_tc_tiling_on_sc=False)


NB = 8


def _segsum_body(C, RPT, hs, srct, dstt, zrows, out, src_v, dst_v, rows,
                 acc, gsem0, gsem1, ssem0, ssem1):
    c = lax.axis_index("c")
    s = lax.axis_index("s")
    wid = c * 16 + s
    gsem = (gsem0, gsem1)
    ssem = (ssem0, ssem1)
    rounds = C // NB

    pltpu.sync_copy(zrows, acc.at[pl.ds(s * RPT, RPT)])
    pltpu.sync_copy(srct.at[wid], src_v)
    pltpu.sync_copy(dstt.at[wid], dst_v)
    plsc.subcore_barrier()

    def fire_gathers(seti, base):
        for b in range(NB):
            pltpu.async_copy(hs.at[src_v.at[base + b]], rows.at[seti, b],
                             gsem[seti])

    def drain_gathers(seti):
        for b in range(NB):
            pltpu.make_async_copy(hs.at[pl.ds(0, K)], rows.at[seti, b],
                                  gsem[seti]).wait()

    def fire_scatters(seti, base):
        for b in range(NB):
            pltpu.async_copy(rows.at[seti, b], acc.at[dst_v.at[base + b]],
                             ssem[seti], add=True)

    def drain_scatters(seti):
        for b in range(NB):
            pltpu.make_async_copy(hs.at[pl.ds(0, K)], rows.at[seti, b],
                                  ssem[seti]).wait()

    fire_gathers(0, 0)
    fire_gathers(1, NB)

    def body(g, carry):
        for i in range(2):
            base = (2 * g + i) * NB
            drain_gathers(i)
            fire_scatters(i, base)
            drain_scatters(i)
            fire_gathers(i, base + 2 * NB)
        return carry

    lax.fori_loop(0, rounds // 2 - 1, body, 0)
    for i in range(2):
        base = (rounds - 2 + i) * NB
        drain_gathers(i)
        fire_scatters(i, base)
        drain_scatters(i)

    plsc.subcore_barrier()
    pltpu.sync_copy(acc.at[pl.ds(s * RPT, RPT)],
                    out.at[c, pl.ds(s * RPT, RPT)])


def _deg_body(C, RPT, dstt, zrows, ones, out, dst_v, ones_v, acc, ssem):
    c = lax.axis_index("c")
    s = lax.axis_index("s")
    wid = c * 16 + s
    pltpu.sync_copy(zrows, acc.at[pl.ds(s * RPT, RPT)])
    pltpu.sync_copy(dstt.at[wid], dst_v)
    pltpu.sync_copy(ones, ones_v)
    plsc.subcore_barrier()

    def fire(base):
        for b in range(8):
            pltpu.async_copy(ones_v, acc.at[dst_v.at[base + b]], ssem,
                             add=True)

    def drain():
        for b in range(8):
            pltpu.make_async_copy(zrows.at[pl.ds(0, K)], ones_v,
                                  ssem).wait()

    fire(0)

    def body(g, carry):
        fire(8 * (g + 1))
        drain()
        return carry

    lax.fori_loop(0, C // 8 - 1, body, 0)
    drain()

    plsc.subcore_barrier()
    pltpu.sync_copy(acc.at[pl.ds(s * RPT, RPT)],
                    out.at[c, pl.ds(s * RPT, RPT)])


def _sc_segsum(hs, srct, dstt, zrows, C, NP, RPT):
    return pl.kernel(
        functools.partial(_segsum_body, C, RPT),
        out_type=jax.ShapeDtypeStruct((2, NP, DH), jnp.float32),
        mesh=_MESH,
        scratch_types=[
            pltpu.VMEM((C, K), jnp.int32),
            pltpu.VMEM((C, K), jnp.int32),
            pltpu.VMEM((2, NB, K, DH), jnp.float32),
            pltpu.VMEM_SHARED((NP, DH), jnp.float32),
            pltpu.SemaphoreType.DMA,
            pltpu.SemaphoreType.DMA,
            pltpu.SemaphoreType.DMA,
            pltpu.SemaphoreType.DMA,
        ],
        compiler_params=_SC_PARAMS,
    )(hs, srct, dstt, zrows)


def _sc_deg(dstt, zrows, ones, C, NP, RPT):
    return pl.kernel(
        functools.partial(_deg_body, C, RPT),
        out_type=jax.ShapeDtypeStruct((2, NP, DH), jnp.float32),
        mesh=_MESH,
        scratch_types=[
            pltpu.VMEM((C, K), jnp.int32),
            pltpu.VMEM((K, DH), jnp.float32),
            pltpu.VMEM_SHARED((NP, DH), jnp.float32),
            pltpu.SemaphoreType.DMA,
        ],
        compiler_params=_SC_PARAMS,
    )(dstt, zrows, ones)


def _edge_body(e_total, n, eb, eidx, s_out, d_out):
    i = pl.program_id(0)
    g = i * eb + lax.broadcasted_iota(jnp.int32, (eb,), 0)
    valid = g < e_total
    pad = n + jnp.remainder(g - e_total, K)
    s_out[:] = jnp.where(valid, eidx[0, :], pad)
    d_out[:] = jnp.where(valid, eidx[1, :], pad)


def _dinv(deg):
    cnt = deg[0, :, 0:1] + deg[1, :, 0:1]
    return lax.rsqrt(cnt + 1.0)


def _tc1_body(x, w1, h_out):
    h_out[:] = jnp.dot(x[:], w1[:], preferred_element_type=jnp.float32)


def _scale_body(deg, h1, out):
    cnt = deg[0] + deg[1]
    out[:] = h1[:] * lax.rsqrt(cnt + 1.0)


def _tc2_body(deg, agg, hs1, b1, w2bd, out):
    cnt = deg[0] + deg[1]
    dv = lax.rsqrt(cnt + 1.0)
    t = dv * (agg[0] + agg[1] + hs1[:]) + b1[0:1, :]
    h = jnp.maximum(t, 0.0)
    out[:] = jnp.dot(h, w2bd[:], preferred_element_type=jnp.float32) * dv


def _tc3_body(deg, agg, hs2, b2, out):
    cnt = deg[0] + deg[1]
    dv = lax.rsqrt(cnt + 1.0)
    t = dv * (agg[0] + agg[1] + hs2[:]) + b2[0:1, :]
    ii = lax.broadcasted_iota(jnp.int32, (128, 128), 0)
    jj = lax.broadcasted_iota(jnp.int32, (128, 128), 1)
    sel = ((jj // DH) * DH == ii).astype(jnp.float32)
    gsum = (jj // DH == ii // DH).astype(jnp.float32)
    m = t
    for k in (1, 2, 4, 8):
        m = jnp.maximum(m, pltpu.roll(m, 128 - k, 1))
    mg = jnp.dot(m, sel, preferred_element_type=jnp.float32)
    sg = jnp.dot(jnp.exp(t - mg), gsum, preferred_element_type=jnp.float32)
    out[:] = t - mg - jnp.log(sg)


def kernel(x, edge_index, W1, b1, W2, b2):
    n, di = x.shape
    e = edge_index.shape[1]
    c_chunks = 8 * (-(-e // (NW * K * 8)))
    epad = NW * c_chunks * K
    rpt = 8 * (-(-(n + K) // (16 * 8)))
    np_rows = 16 * rpt

    eb = epad // 8
    sflat, dflat = pl.pallas_call(
        functools.partial(_edge_body, e, n, eb),
        grid=(8,),
        in_specs=[pl.BlockSpec((2, eb), lambda i: (0, i))],
        out_specs=[pl.BlockSpec((eb,), lambda i: (i,)),
                   pl.BlockSpec((eb,), lambda i: (i,))],
        out_shape=[jax.ShapeDtypeStruct((epad,), jnp.int32),
                   jax.ShapeDtypeStruct((epad,), jnp.int32)],
    )(edge_index)
    srct = sflat.reshape(NW, c_chunks, K)
    dstt = dflat.reshape(NW, c_chunks, K)
    zrows = jnp.zeros((rpt, DH), jnp.float32)
    ones = jnp.ones((K, DH), jnp.float32)

    deg = _sc_deg(dstt, zrows, ones, c_chunks, np_rows, rpt)

    blk = 2000
    grid = (n // blk,)
    row16 = lambda i: (i, 0)
    h1 = pl.pallas_call(
        _tc1_body,
        grid=grid,
        in_specs=[
            pl.BlockSpec((blk, di), row16),
            pl.BlockSpec((di, DH), lambda i: (0, 0)),
        ],
        out_specs=pl.BlockSpec((blk, DH), row16),
        out_shape=jax.ShapeDtypeStruct((np_rows, DH), jnp.float32),
    )(x, W1)

    npk = np_rows // 8
    blkp = npk // grid[0]
    degp = deg.reshape(2, npk, 128)
    h1p = h1.reshape(npk, 128)
    pk_spec = pl.BlockSpec((blkp, 128), row16)
    pk2_spec = pl.BlockSpec((2, blkp, 128), lambda i: (0, i, 0))

    hs1p = pl.pallas_call(
        _scale_body,
        grid=grid,
        in_specs=[pk2_spec, pk_spec],
        out_specs=pk_spec,
        out_shape=jax.ShapeDtypeStruct((npk, 128), jnp.float32),
    )(degp, h1p)

    agg1 = _sc_segsum(hs1p.reshape(np_rows, DH), srct, dstt, zrows,
                      c_chunks, np_rows, rpt)
    agg1p = agg1.reshape(2, npk, 128)
    b1p = jnp.broadcast_to(jnp.tile(b1, 8).reshape(1, 128), (8, 128))
    b2p = jnp.broadcast_to(jnp.tile(b2, 8).reshape(1, 128), (8, 128))
    w2bd = jnp.kron(jnp.eye(8, dtype=jnp.float32), W2)

    hs2p = pl.pallas_call(
        _tc2_body,
        grid=grid,
        in_specs=[
            pk2_spec,
            pk2_spec,
            pk_spec,
            pl.BlockSpec((8, 128), lambda i: (0, 0)),
            pl.BlockSpec((128, 128), lambda i: (0, 0)),
        ],
        out_specs=pk_spec,
        out_shape=jax.ShapeDtypeStruct((npk, 128), jnp.float32),
    )(degp, agg1p, hs1p, b1p, w2bd)

    agg2 = _sc_segsum(hs2p.reshape(np_rows, DH), srct, dstt, zrows,
                      c_chunks, np_rows, rpt)

    outp = pl.pallas_call(
        _tc3_body,
        grid=grid,
        in_specs=[
            pk2_spec,
            pk2_spec,
            pk_spec,
            pl.BlockSpec((8, 128), lambda i: (0, 0)),
        ],
        out_specs=pk_spec,
        out_shape=jax.ShapeDtypeStruct((npk, 128), jnp.float32),
    )(degp, agg2.reshape(2, npk, 128), hs2p, b2p)

    return outp.reshape(np_rows, DH)[:n]
